# baseline (speedup 1.0000x reference)
import functools

import jax
import jax.numpy as jnp
from jax import lax
from jax.experimental import pallas as pl
from jax.experimental.pallas import tpu as pltpu
from jax.experimental.pallas import tpu_sc as plsc

_VOCAB = 1000000
_EMBED = 64
_BATCH = 4096
_SEQ = 200

_INFO = plsc.get_sparse_core_info()
_NC = _INFO.num_cores
_NS = _INFO.num_subcores
_NW = _NC * _NS

_BW = _BATCH // _NW
_NBUF = 4

_MESH = plsc.VectorSubcoreMesh(core_axis_name="c", subcore_axis_name="s")


@functools.partial(
    pl.kernel,
    out_type=jax.ShapeDtypeStruct((_SEQ, _BATCH, 2 * _EMBED), jnp.float32),
    mesh=_MESH,
    compiler_params=pltpu.CompilerParams(
        use_tc_tiling_on_sc=False, needs_layout_passes=False),
    scratch_types=[
        pltpu.VMEM((_SEQ, _BW), jnp.int32),
        pltpu.VMEM((_NBUF, _BW, _EMBED), jnp.float32),
        pltpu.SemaphoreType.DMA,
        pltpu.SemaphoreType.DMA,
        pltpu.SemaphoreType.DMA,
        pltpu.SemaphoreType.DMA,
        pltpu.SemaphoreType.DMA,
        pltpu.SemaphoreType.DMA,
        pltpu.SemaphoreType.DMA,
        pltpu.SemaphoreType.DMA,
    ],
)
def _embed_kernel(idx_hbm, table_hbm, out_hbm, idx_v, rows_v,
                  g0, g1, g2, g3, s0, s1, s2, s3):
    wid = lax.axis_index("s") * _NC + lax.axis_index("c")
    b0 = wid * _BW
    gsem = (g0, g1, g2, g3)
    ssem = (s0, s1, s2, s3)

    pltpu.sync_copy(idx_hbm.at[:, pl.ds(b0, _BW)], idx_v)

    def scale_step(s, carry):
        for j in range(_BW // 16):
            v = idx_v[s, pl.ds(16 * j, 16)]
            idx_v[s, pl.ds(16 * j, 16)] = v + v
        return carry

    lax.fori_loop(0, _SEQ, scale_step, 0)

    def gather_start(s, b):
        pltpu.async_copy(table_hbm.at[idx_v.at[s]], rows_v.at[b], gsem[b])

    def gather_wait(s, b):
        pltpu.make_async_copy(
            table_hbm.at[idx_v.at[s]], rows_v.at[b], gsem[b]).wait()

    def store_start(s, b):
        pltpu.async_copy(
            rows_v.at[b], out_hbm.at[s, pl.ds(b0, _BW), pl.ds(0, _EMBED)],
            ssem[b])

    def store_wait(s, b):
        pltpu.make_async_copy(
            rows_v.at[b], out_hbm.at[s, pl.ds(b0, _BW), pl.ds(0, _EMBED)],
            ssem[b]).wait()

    for b in range(_NBUF):
        gather_start(b, b)

    def step(k, carry):
        for b in range(_NBUF):
            s = k * _NBUF + b
            gather_wait(s, b)
            store_start(s, b)
            store_wait(s, b)

            @pl.when(s + _NBUF < _SEQ)
            def _():
                gather_start(s + _NBUF, b)

        return carry

    lax.fori_loop(0, _SEQ // _NBUF, step, 0)


_WBLK = 512


def _wprep_body(w_ref, out_ref):
    x = w_ref[...].T
    out_ref[...] = jnp.concatenate([x, x], axis=1)


_wprep = pl.pallas_call(
    _wprep_body,
    grid=((_VOCAB + _WBLK - 1) // _WBLK,),
    in_specs=[pl.BlockSpec((_EMBED, _WBLK), lambda i: (0, i))],
    out_specs=pl.BlockSpec((_WBLK, 2 * _EMBED), lambda i: (i, 0)),
    out_shape=jax.ShapeDtypeStruct((_VOCAB, 2 * _EMBED), jnp.float32),
)


def kernel(input, weight):
    idx_t = input.T.astype(jnp.int32)
    table = _wprep(weight.T).reshape(2 * _VOCAB, _EMBED)
    out = _embed_kernel(idx_t, table)
    return out[:, :, :_EMBED]


# speedup vs baseline: 1.4085x; 1.4085x over previous
import functools

import jax
import jax.numpy as jnp
from jax import lax
from jax.experimental import pallas as pl
from jax.experimental.pallas import tpu as pltpu
from jax.experimental.pallas import tpu_sc as plsc

_VOCAB = 1000000
_EMBED = 64
_BATCH = 4096
_SEQ = 200

_INFO = plsc.get_sparse_core_info()
_NC = _INFO.num_cores
_NS = _INFO.num_subcores
_NW = _NC * _NS

_BW = _BATCH // _NW
_NBUF = 4

_MESH = plsc.VectorSubcoreMesh(core_axis_name="c", subcore_axis_name="s")


@functools.partial(
    pl.kernel,
    out_type=jax.ShapeDtypeStruct((_SEQ, _BATCH, 2 * _EMBED), jnp.float32),
    mesh=_MESH,
    compiler_params=pltpu.CompilerParams(
        use_tc_tiling_on_sc=False, needs_layout_passes=False),
    scratch_types=[
        pltpu.VMEM((_SEQ, _BW), jnp.int32),
        pltpu.VMEM((_NBUF, _BW, _EMBED), jnp.float32),
        pltpu.SemaphoreType.DMA,
        pltpu.SemaphoreType.DMA,
        pltpu.SemaphoreType.DMA,
        pltpu.SemaphoreType.DMA,
        pltpu.SemaphoreType.DMA,
        pltpu.SemaphoreType.DMA,
        pltpu.SemaphoreType.DMA,
        pltpu.SemaphoreType.DMA,
    ],
)
def _embed_kernel(idx_hbm, table_hbm, out_hbm, idx_v, rows_v,
                  g0, g1, g2, g3, s0, s1, s2, s3):
    wid = lax.axis_index("s") * _NC + lax.axis_index("c")
    b0 = wid * _BW
    gsem = (g0, g1, g2, g3)
    ssem = (s0, s1, s2, s3)

    pltpu.sync_copy(idx_hbm.at[:, pl.ds(b0, _BW)], idx_v)

    def scale_step(s, carry):
        for j in range(_BW // 16):
            v = idx_v[s, pl.ds(16 * j, 16)]
            idx_v[s, pl.ds(16 * j, 16)] = v + v
        return carry

    lax.fori_loop(0, _SEQ, scale_step, 0)

    def gather_start(s, b):
        pltpu.async_copy(table_hbm.at[idx_v.at[s]], rows_v.at[b], gsem[b])

    def gather_wait(s, b):
        pltpu.make_async_copy(
            table_hbm.at[idx_v.at[s]], rows_v.at[b], gsem[b]).wait()

    def store_start(s, b):
        pltpu.async_copy(
            rows_v.at[b], out_hbm.at[s, pl.ds(b0, _BW), pl.ds(0, _EMBED)],
            ssem[b])

    def store_wait(s, b):
        pltpu.make_async_copy(
            rows_v.at[b], out_hbm.at[s, pl.ds(b0, _BW), pl.ds(0, _EMBED)],
            ssem[b]).wait()

    for b in range(_NBUF):
        gather_start(b, b)

    def step(k, carry):
        for b in range(_NBUF):
            s = k * _NBUF + b
            gather_wait(s, b)
            store_start(s, b)
            store_wait(s, b)

            @pl.when(s + _NBUF < _SEQ)
            def _():
                gather_start(s + _NBUF, b)

        return carry

    lax.fori_loop(0, _SEQ // _NBUF, step, 0)


_WBLK = 1024


def _wprep_body(w_ref, out_ref):
    x = w_ref[...]
    eye = (lax.broadcasted_iota(jnp.int32, (_EMBED, _EMBED), 0)
           == lax.broadcasted_iota(jnp.int32, (_EMBED, _EMBED), 1)
           ).astype(jnp.float32)
    xt = lax.dot_general(x, eye, (((0,), (0,)), ((), ())),
                         preferred_element_type=jnp.float32)
    out_ref[...] = jnp.concatenate([xt, xt], axis=1)


_wprep = pl.pallas_call(
    _wprep_body,
    grid=((_VOCAB + _WBLK - 1) // _WBLK,),
    in_specs=[pl.BlockSpec((_EMBED, _WBLK), lambda i: (0, i))],
    out_specs=pl.BlockSpec((_WBLK, 2 * _EMBED), lambda i: (i, 0)),
    out_shape=jax.ShapeDtypeStruct((_VOCAB, 2 * _EMBED), jnp.float32),
)


def kernel(input, weight):
    idx_t = input.T.astype(jnp.int32)
    table = _wprep(weight.T).reshape(2 * _VOCAB, _EMBED)
    out = _embed_kernel(idx_t, table)
    return out[:, :, :_EMBED]


# speedup vs baseline: 1.9679x; 1.3971x over previous
import functools

import jax
import jax.numpy as jnp
from jax import lax
from jax.experimental import pallas as pl
from jax.experimental.pallas import tpu as pltpu
from jax.experimental.pallas import tpu_sc as plsc

_VOCAB = 1000000
_EMBED = 64
_BATCH = 4096
_SEQ = 200

_INFO = plsc.get_sparse_core_info()
_NC = _INFO.num_cores
_NS = _INFO.num_subcores
_NW = _NC * _NS

_BW = _BATCH // _NW
_NBUF = 4

_MESH = plsc.VectorSubcoreMesh(core_axis_name="c", subcore_axis_name="s")


@functools.partial(
    pl.kernel,
    out_type=jax.ShapeDtypeStruct((_SEQ, _BATCH, 2 * _EMBED), jnp.float32),
    mesh=_MESH,
    compiler_params=pltpu.CompilerParams(
        use_tc_tiling_on_sc=False, needs_layout_passes=False),
    scratch_types=[
        pltpu.VMEM((_SEQ, _BW), jnp.int32),
        pltpu.VMEM((_NBUF, _BW, _EMBED), jnp.float32),
        pltpu.SemaphoreType.DMA,
        pltpu.SemaphoreType.DMA,
        pltpu.SemaphoreType.DMA,
        pltpu.SemaphoreType.DMA,
        pltpu.SemaphoreType.DMA,
        pltpu.SemaphoreType.DMA,
        pltpu.SemaphoreType.DMA,
        pltpu.SemaphoreType.DMA,
    ],
)
def _embed_kernel(idx_hbm, table_hbm, out_hbm, idx_v, rows_v,
                  g0, g1, g2, g3, s0, s1, s2, s3):
    wid = lax.axis_index("s") * _NC + lax.axis_index("c")
    b0 = wid * _BW
    gsem = (g0, g1, g2, g3)
    ssem = (s0, s1, s2, s3)

    pltpu.sync_copy(idx_hbm.at[:, pl.ds(b0, _BW)], idx_v)

    def scale_step(s, carry):
        for j in range(_BW // 16):
            v = idx_v[s, pl.ds(16 * j, 16)]
            idx_v[s, pl.ds(16 * j, 16)] = v + v
        return carry

    lax.fori_loop(0, _SEQ, scale_step, 0)

    def gather_start(s, b):
        pltpu.async_copy(table_hbm.at[idx_v.at[s]], rows_v.at[b], gsem[b])

    def gather_wait(s, b):
        pltpu.make_async_copy(
            table_hbm.at[idx_v.at[s]], rows_v.at[b], gsem[b]).wait()

    def store_start(s, b):
        pltpu.async_copy(
            rows_v.at[b], out_hbm.at[s, pl.ds(b0, _BW), pl.ds(0, _EMBED)],
            ssem[b])

    def store_wait(s, b):
        pltpu.make_async_copy(
            rows_v.at[b], out_hbm.at[s, pl.ds(b0, _BW), pl.ds(0, _EMBED)],
            ssem[b]).wait()

    for b in range(_NBUF):
        gather_start(b, b)

    def step(k, carry):
        for b in range(_NBUF):
            s = k * _NBUF + b
            gather_wait(s, b)
            store_start(s, b)
            store_wait(s, b)

            @pl.when(s + _NBUF < _SEQ)
            def _():
                gather_start(s + _NBUF, b)

        return carry

    lax.fori_loop(0, _SEQ // _NBUF, step, 0)


_WBLK = 4096


def _wprep_body(w_ref, out_ref):
    x = w_ref[...]
    eye = (lax.broadcasted_iota(jnp.int32, (_EMBED, _EMBED), 0)
           == lax.broadcasted_iota(jnp.int32, (_EMBED, _EMBED), 1)
           ).astype(jnp.float32)
    xt = lax.dot_general(x, eye, (((0,), (0,)), ((), ())),
                         precision=lax.Precision.HIGHEST,
                         preferred_element_type=jnp.float32)
    out_ref[...] = jnp.concatenate([xt, xt], axis=1)


_wprep = pl.pallas_call(
    _wprep_body,
    grid=((_VOCAB + _WBLK - 1) // _WBLK,),
    in_specs=[pl.BlockSpec((_EMBED, _WBLK), lambda i: (0, i))],
    out_specs=pl.BlockSpec((_WBLK, 2 * _EMBED), lambda i: (i, 0)),
    out_shape=jax.ShapeDtypeStruct((_VOCAB, 2 * _EMBED), jnp.float32),
)


def kernel(input, weight):
    idx_t = input.T.astype(jnp.int32)
    table = _wprep(weight.T).reshape(2 * _VOCAB, _EMBED)
    out = _embed_kernel(idx_t, table)
    return out[:, :, :_EMBED]


# speedup vs baseline: 2.0820x; 1.0580x over previous
import functools

import jax
import jax.numpy as jnp
from jax import lax
from jax.experimental import pallas as pl
from jax.experimental.pallas import tpu as pltpu
from jax.experimental.pallas import tpu_sc as plsc

_VOCAB = 1000000
_EMBED = 64
_BATCH = 4096
_SEQ = 200

_INFO = plsc.get_sparse_core_info()
_NC = _INFO.num_cores
_NS = _INFO.num_subcores
_NW = _NC * _NS

_BW = _BATCH // _NW
_NBUF = 4

_MESH = plsc.VectorSubcoreMesh(core_axis_name="c", subcore_axis_name="s")


@functools.partial(
    pl.kernel,
    out_type=jax.ShapeDtypeStruct((_SEQ, _BATCH, 2 * _EMBED), jnp.float32),
    mesh=_MESH,
    compiler_params=pltpu.CompilerParams(
        use_tc_tiling_on_sc=False, needs_layout_passes=False),
    scratch_types=[
        pltpu.VMEM((_SEQ, _BW), jnp.int32),
        pltpu.VMEM((_NBUF, _BW, _EMBED), jnp.float32),
        pltpu.SemaphoreType.DMA,
        pltpu.SemaphoreType.DMA,
        pltpu.SemaphoreType.DMA,
        pltpu.SemaphoreType.DMA,
        pltpu.SemaphoreType.DMA,
        pltpu.SemaphoreType.DMA,
        pltpu.SemaphoreType.DMA,
        pltpu.SemaphoreType.DMA,
    ],
)
def _embed_kernel(idx_hbm, table_hbm, out_hbm, idx_v, rows_v,
                  g0, g1, g2, g3, s0, s1, s2, s3):
    wid = lax.axis_index("s") * _NC + lax.axis_index("c")
    b0 = wid * _BW
    gsem = (g0, g1, g2, g3)
    ssem = (s0, s1, s2, s3)

    pltpu.sync_copy(idx_hbm.at[:, pl.ds(b0, _BW)], idx_v)

    def scale_step(s, carry):
        for j in range(_BW // 16):
            v = idx_v[s, pl.ds(16 * j, 16)]
            idx_v[s, pl.ds(16 * j, 16)] = v + v
        return carry

    lax.fori_loop(0, _SEQ, scale_step, 0)

    def gather_start(s, b):
        pltpu.async_copy(table_hbm.at[idx_v.at[s]], rows_v.at[b], gsem[b])

    def gather_wait(s, b):
        pltpu.make_async_copy(
            table_hbm.at[idx_v.at[s]], rows_v.at[b], gsem[b]).wait()

    def store_start(s, b):
        pltpu.async_copy(
            rows_v.at[b], out_hbm.at[s, pl.ds(b0, _BW), pl.ds(0, _EMBED)],
            ssem[b])

    def store_wait(s, b):
        pltpu.make_async_copy(
            rows_v.at[b], out_hbm.at[s, pl.ds(b0, _BW), pl.ds(0, _EMBED)],
            ssem[b]).wait()

    for b in range(_NBUF):
        gather_start(b, b)

    def step(k, carry):
        for b in range(_NBUF):
            s = k * _NBUF + b
            gather_wait(s, b)
            store_start(s, b)
            store_wait(s, b)

            @pl.when(s + _NBUF < _SEQ)
            def _():
                gather_start(s + _NBUF, b)

        return carry

    lax.fori_loop(0, _SEQ // _NBUF, step, 0)


_WBLK = 8192


def _wprep_body(w_ref, out_ref):
    x = w_ref[...]
    eye = (lax.broadcasted_iota(jnp.int32, (_EMBED, _EMBED), 0)
           == lax.broadcasted_iota(jnp.int32, (_EMBED, _EMBED), 1)
           ).astype(jnp.float32)
    xt = lax.dot_general(x, eye, (((0,), (0,)), ((), ())),
                         precision=lax.Precision.HIGHEST,
                         preferred_element_type=jnp.float32)
    out_ref[...] = jnp.concatenate([xt, xt], axis=1)


_wprep = pl.pallas_call(
    _wprep_body,
    grid=((_VOCAB + _WBLK - 1) // _WBLK,),
    in_specs=[pl.BlockSpec((_EMBED, _WBLK), lambda i: (0, i))],
    out_specs=pl.BlockSpec((_WBLK, 2 * _EMBED), lambda i: (i, 0)),
    out_shape=jax.ShapeDtypeStruct((_VOCAB, 2 * _EMBED), jnp.float32),
)


def kernel(input, weight):
    idx_t = input.T.astype(jnp.int32)
    table = _wprep(weight.T).reshape(2 * _VOCAB, _EMBED)
    out = _embed_kernel(idx_t, table)
    return out[:, :, :_EMBED]
